# baseline (speedup 1.0000x reference)
import functools

import jax
import jax.numpy as jnp
from jax.experimental import pallas as pl
from jax.experimental.pallas import tpu as pltpu

_SQRT_2_OVER_PI = 0.7978845608028654

_RT = 8
_BM = 32
_WAVE = 8


def _mlp_kernel(x_ref, w1_ref, b1_ref, w2_ref, b2_ref, o_ref, *,
                bm, rt, p, ob, q):
    w1 = w1_ref[...]
    b1 = b1_ref[...]
    w2 = w2_ref[...]
    b2 = b2_ref[...]
    c1 = jnp.bfloat16(_SQRT_2_OVER_PI)
    c2 = jnp.bfloat16(0.044715 * _SQRT_2_OVER_PI)
    half = jnp.bfloat16(0.5)
    wave = min(_WAVE, bm)
    for c0 in range(0, bm, wave):
        hs = []
        for c in range(c0, c0 + wave):
            xc = (x_ref[c * rt:(c + 1) * rt]
                  .reshape(rt * p, q).astype(jnp.bfloat16))
            hs.append(jnp.dot(w1, xc, preferred_element_type=jnp.float32)
                      .astype(jnp.bfloat16) + b1)
        gs = []
        for h in hs:
            h2 = h * h
            t = jnp.tanh(h * (c1 + c2 * h2))
            u = half * h
            gs.append(u + u * t)
        for j, c in enumerate(range(c0, c0 + wave)):
            o = jnp.dot(w2, gs[j], preferred_element_type=jnp.float32) + b2
            o_ref[c * rt:(c + 1) * rt] = o.reshape(rt, ob, q)


def _kron_eye(w, n):
    a, b = w.shape
    eye = jnp.eye(n, dtype=w.dtype)
    return (eye[:, None, :, None] * w[None, :, None, :]).reshape(n * a, n * b)


def kernel(x, w1, b1, w2, b2):
    B, A, P, Q = x.shape
    h1, cin = w1.shape
    s = P // cin
    co = w2.shape[0]
    assert P == cin * s and Q % 128 == 0
    R = B * A
    rt, bm = _RT, _BM
    rows_per_step = bm * rt
    assert R % rows_per_step == 0

    x3 = x.reshape(R, P, Q)

    eye_s = jnp.eye(s, dtype=jnp.float32)
    w1f = (w1[None, :, :, None] * eye_s[:, None, None, :]).reshape(
        s * h1, s * cin)
    w2f = (eye_s[:, None, :, None] * w2[None, :, None, :]).reshape(
        s * co, s * h1)
    hb, ob = s * h1, s * co
    w1bd = _kron_eye(w1f, rt).astype(jnp.bfloat16)
    w2bd = _kron_eye(w2f, rt).astype(jnp.bfloat16)
    b1bd = jnp.tile(jnp.tile(b1, s), rt).reshape(rt * hb, 1).astype(
        jnp.bfloat16)
    b2bd = jnp.tile(jnp.tile(b2, s), rt).reshape(rt * ob, 1).astype(
        jnp.float32)

    kfn = functools.partial(_mlp_kernel, bm=bm, rt=rt, p=P, ob=ob, q=Q)
    flops = int(2 * R * Q * (hb * P + ob * hb))
    cost = pl.CostEstimate(
        flops=flops,
        transcendentals=int(R * Q * hb),
        bytes_accessed=int(x.size * 4 + R * ob * Q * 4))

    y = pl.pallas_call(
        kfn,
        out_shape=jax.ShapeDtypeStruct((R, ob, Q), x.dtype),
        grid=(R // rows_per_step,),
        in_specs=[
            pl.BlockSpec((rows_per_step, P, Q), lambda i: (i, 0, 0)),
            pl.BlockSpec((rt * hb, rt * P), lambda i: (0, 0)),
            pl.BlockSpec((rt * hb, 1), lambda i: (0, 0)),
            pl.BlockSpec((rt * ob, rt * hb), lambda i: (0, 0)),
            pl.BlockSpec((rt * ob, 1), lambda i: (0, 0)),
        ],
        out_specs=pl.BlockSpec((rows_per_step, ob, Q),
                               lambda i: (i, 0, 0)),
        compiler_params=pltpu.CompilerParams(
            dimension_semantics=("parallel",),
            vmem_limit_bytes=32 * 1024 * 1024),
        cost_estimate=cost,
    )(x3, w1bd, b1bd, w2bd, b2bd)
    return y.reshape(B, A, ob, Q)


# speedup vs baseline: 1.0142x; 1.0142x over previous
import functools

import jax
import jax.numpy as jnp
from jax.experimental import pallas as pl
from jax.experimental.pallas import tpu as pltpu

_SQRT_2_OVER_PI = 0.7978845608028654

_RT = 8
_BM = 64
_WAVE = 16


def _mlp_kernel(x_ref, w1_ref, b1_ref, w2_ref, b2_ref, o_ref, *,
                bm, rt, p, ob, q):
    w1 = w1_ref[...]
    b1 = b1_ref[...]
    w2 = w2_ref[...]
    b2 = b2_ref[...]
    c1 = jnp.bfloat16(_SQRT_2_OVER_PI)
    c2 = jnp.bfloat16(0.044715 * _SQRT_2_OVER_PI)
    half = jnp.bfloat16(0.5)
    wave = min(_WAVE, bm)
    for c0 in range(0, bm, wave):
        hs = []
        for c in range(c0, c0 + wave):
            xc = (x_ref[c * rt:(c + 1) * rt]
                  .reshape(rt * p, q).astype(jnp.bfloat16))
            hs.append(jnp.dot(w1, xc, preferred_element_type=jnp.float32)
                      .astype(jnp.bfloat16) + b1)
        gs = []
        for h in hs:
            h2 = h * h
            t = jnp.tanh(h * (c1 + c2 * h2))
            u = half * h
            gs.append(u + u * t)
        for j, c in enumerate(range(c0, c0 + wave)):
            o = jnp.dot(w2, gs[j], preferred_element_type=jnp.float32) + b2
            o_ref[c * rt:(c + 1) * rt] = o.reshape(rt, ob, q)


def _kron_eye(w, n):
    a, b = w.shape
    eye = jnp.eye(n, dtype=w.dtype)
    return (eye[:, None, :, None] * w[None, :, None, :]).reshape(n * a, n * b)


def kernel(x, w1, b1, w2, b2):
    B, A, P, Q = x.shape
    h1, cin = w1.shape
    s = P // cin
    co = w2.shape[0]
    assert P == cin * s and Q % 128 == 0
    R = B * A
    rt, bm = _RT, _BM
    rows_per_step = bm * rt
    assert R % rows_per_step == 0

    x3 = x.reshape(R, P, Q)

    eye_s = jnp.eye(s, dtype=jnp.float32)
    w1f = (w1[None, :, :, None] * eye_s[:, None, None, :]).reshape(
        s * h1, s * cin)
    w2f = (eye_s[:, None, :, None] * w2[None, :, None, :]).reshape(
        s * co, s * h1)
    hb, ob = s * h1, s * co
    w1bd = _kron_eye(w1f, rt).astype(jnp.bfloat16)
    w2bd = _kron_eye(w2f, rt).astype(jnp.bfloat16)
    b1bd = jnp.tile(jnp.tile(b1, s), rt).reshape(rt * hb, 1).astype(
        jnp.bfloat16)
    b2bd = jnp.tile(jnp.tile(b2, s), rt).reshape(rt * ob, 1).astype(
        jnp.float32)

    kfn = functools.partial(_mlp_kernel, bm=bm, rt=rt, p=P, ob=ob, q=Q)
    flops = int(2 * R * Q * (hb * P + ob * hb))
    cost = pl.CostEstimate(
        flops=flops,
        transcendentals=int(R * Q * hb),
        bytes_accessed=int(x.size * 4 + R * ob * Q * 4))

    y = pl.pallas_call(
        kfn,
        out_shape=jax.ShapeDtypeStruct((R, ob, Q), x.dtype),
        grid=(R // rows_per_step,),
        in_specs=[
            pl.BlockSpec((rows_per_step, P, Q), lambda i: (i, 0, 0)),
            pl.BlockSpec((rt * hb, rt * P), lambda i: (0, 0)),
            pl.BlockSpec((rt * hb, 1), lambda i: (0, 0)),
            pl.BlockSpec((rt * ob, rt * hb), lambda i: (0, 0)),
            pl.BlockSpec((rt * ob, 1), lambda i: (0, 0)),
        ],
        out_specs=pl.BlockSpec((rows_per_step, ob, Q),
                               lambda i: (i, 0, 0)),
        compiler_params=pltpu.CompilerParams(
            dimension_semantics=("parallel",),
            vmem_limit_bytes=32 * 1024 * 1024),
        cost_estimate=cost,
    )(x3, w1bd, b1bd, w2bd, b2bd)
    return y.reshape(B, A, ob, Q)


# speedup vs baseline: 1.2039x; 1.1871x over previous
import functools

import jax
import jax.numpy as jnp
from jax.experimental import pallas as pl
from jax.experimental.pallas import tpu as pltpu

_SQRT_2_OVER_PI = 0.7978845608028654

_RT = 8
_BM = 64
_WAVE = 16


def _mlp_kernel(x_ref, w1_ref, b1_ref, w2_ref, b2_ref, o_ref, *,
                bm, rt, cin, h1, co, q, s):
    w1 = w1_ref[...]
    b1 = b1_ref[...]
    w2 = w2_ref[...]
    b2 = b2_ref[...]
    c1 = jnp.bfloat16(_SQRT_2_OVER_PI)
    c2 = jnp.bfloat16(0.044715 * _SQRT_2_OVER_PI)
    half = jnp.bfloat16(0.5)
    wave = min(_WAVE, bm)
    for c0 in range(0, bm, wave):
        hs = []
        for c in range(c0, c0 + wave):
            xg = jnp.concatenate(
                [x_ref[c * rt:(c + 1) * rt, i::s, :] for i in range(s)],
                axis=2)
            xc = xg.reshape(rt * cin, s * q).astype(jnp.bfloat16)
            hs.append(jnp.dot(w1, xc, preferred_element_type=jnp.float32)
                      .astype(jnp.bfloat16) + b1)
        gs = []
        for h in hs:
            h2 = h * h
            t = jnp.tanh(h * (c1 + c2 * h2))
            u = half * h
            gs.append(u + u * t)
        for j, c in enumerate(range(c0, c0 + wave)):
            o = jnp.dot(w2, gs[j], preferred_element_type=jnp.float32) + b2
            o3 = o.reshape(rt, co, s * q)
            for i in range(s):
                o_ref[c * rt:(c + 1) * rt, i * co:(i + 1) * co, :] = (
                    o3[:, :, i * q:(i + 1) * q])


def _kron_eye(w, n):
    a, b = w.shape
    eye = jnp.eye(n, dtype=w.dtype)
    return (eye[:, None, :, None] * w[None, :, None, :]).reshape(n * a, n * b)


def kernel(x, w1, b1, w2, b2):
    B, A, P, Q = x.shape
    h1, cin = w1.shape
    s = P // cin
    co = w2.shape[0]
    assert P == cin * s and Q % 128 == 0
    R = B * A
    rt, bm = _RT, _BM
    rows_per_step = bm * rt
    assert R % rows_per_step == 0

    x3 = x.reshape(R, P, Q)

    ob = s * co
    w1bd = _kron_eye(w1, rt).astype(jnp.bfloat16)
    w2bd = _kron_eye(w2, rt).astype(jnp.bfloat16)
    b1bd = jnp.tile(b1, rt).reshape(rt * h1, 1).astype(jnp.bfloat16)
    b2bd = jnp.tile(b2, rt).reshape(rt * co, 1).astype(jnp.float32)

    kfn = functools.partial(_mlp_kernel, bm=bm, rt=rt, cin=cin, h1=h1,
                            co=co, q=Q, s=s)
    flops = int(2 * R * Q * s * (h1 * cin + co * h1))
    cost = pl.CostEstimate(
        flops=flops,
        transcendentals=int(R * Q * s * h1),
        bytes_accessed=int(x.size * 4 + R * ob * Q * 4))

    y = pl.pallas_call(
        kfn,
        out_shape=jax.ShapeDtypeStruct((R, ob, Q), x.dtype),
        grid=(R // rows_per_step,),
        in_specs=[
            pl.BlockSpec((rows_per_step, P, Q), lambda i: (i, 0, 0)),
            pl.BlockSpec((rt * h1, rt * cin), lambda i: (0, 0)),
            pl.BlockSpec((rt * h1, 1), lambda i: (0, 0)),
            pl.BlockSpec((rt * co, rt * h1), lambda i: (0, 0)),
            pl.BlockSpec((rt * co, 1), lambda i: (0, 0)),
        ],
        out_specs=pl.BlockSpec((rows_per_step, ob, Q),
                               lambda i: (i, 0, 0)),
        compiler_params=pltpu.CompilerParams(
            dimension_semantics=("parallel",),
            vmem_limit_bytes=32 * 1024 * 1024),
        cost_estimate=cost,
    )(x3, w1bd, b1bd, w2bd, b2bd)
    return y.reshape(B, A, ob, Q)
